# baseline (speedup 1.0000x reference)
import functools

import jax
import jax.numpy as jnp
from jax import lax
from jax.experimental import pallas as pl
from jax.experimental.pallas import tpu as pltpu


def _im2col9(xf, NI, H, W):
    M, Cin = xf.shape
    HW = H * W
    dt = xf.dtype
    p = lax.broadcasted_iota(jnp.int32, (M, 1), 0)
    j = p % W
    pin = p % HW
    taps = []
    for kh in range(3):
        for kw in range(3):
            t = (kh - 1) * W + (kw - 1)
            if t > 0:
                sh = jnp.concatenate([xf[t:], jnp.zeros((t, Cin), dt)], axis=0)
            elif t < 0:
                sh = jnp.concatenate([jnp.zeros((-t, Cin), dt), xf[:t]], axis=0)
            else:
                sh = xf
            mask = None
            if kw == 0:
                mask = j != 0
            elif kw == 2:
                mask = j != W - 1
            if kh == 0:
                vm = pin >= W
                mask = vm if mask is None else (mask & vm)
            elif kh == 2:
                vm = pin < HW - W
                mask = vm if mask is None else (mask & vm)
            if mask is not None:
                sh = jnp.where(mask, sh, jnp.zeros((), dt))
            taps.append(sh)
    return jnp.concatenate(taps, axis=-1)


def _block_kernel(x_ref, w1_ref, s1_ref, b1_ref, mc_ref, w2_ref, s2_ref,
                  b2_ref, m2_ref, wfc1_ref, sf1_ref, bf1_ref, wfc2_ref,
                  bfc2_ref, o_ref, *, NI, H, W, C):
    HW = H * W
    M = NI * HW
    Cq = C // 2
    xf = x_ref[...].reshape(M, x_ref.shape[-1])

    TM = 1024
    cc1 = _im2col9(xf, NI, H, W)
    acc1 = jnp.concatenate(
        [jnp.dot(cc1[t:t + TM], w1_ref[...], preferred_element_type=jnp.float32)
         for t in range(0, M, TM)], axis=0)
    y = acc1 * s1_ref[...] + b1_ref[...]
    y = jnp.where(mc_ref[...] > 0.0, jnp.maximum(y, 0.0), y)
    y1 = y[:, :C].astype(jnp.bfloat16)
    res = y[:, C:]

    cc2 = _im2col9(y1, NI, H, W)
    acc2 = jnp.concatenate(
        [jnp.dot(cc2[t:t + TM], w2_ref[...], preferred_element_type=jnp.float32)
         for t in range(0, M, TM)], axis=0)
    x2 = acc2 * s2_ref[...] + b2_ref[...]
    x2 = jnp.where(m2_ref[...] > 0.0, jnp.maximum(x2, 0.0), x2)

    gsum = jnp.sum(x2.reshape(NI, HW, 2 * C), axis=1)
    gap = (gsum[:, :C] + gsum[:, C:]) * (1.0 / HW)
    g1 = jnp.dot(gap, wfc1_ref[...], preferred_element_type=jnp.float32)
    g1 = jnp.maximum(g1 * sf1_ref[...] + bf1_ref[...], 0.0)
    a = jnp.dot(g1, wfc2_ref[...], preferred_element_type=jnp.float32)
    a = a + bfc2_ref[...]

    a00, a01 = a[:, 0:Cq], a[:, Cq:2 * Cq]
    a10, a11 = a[:, 2 * Cq:3 * Cq], a[:, 3 * Cq:4 * Cq]
    m0 = jnp.maximum(a00, a01)
    e00, e01 = jnp.exp(a00 - m0), jnp.exp(a01 - m0)
    r0 = 1.0 / (e00 + e01)
    m1 = jnp.maximum(a10, a11)
    e10, e11 = jnp.exp(a10 - m1), jnp.exp(a11 - m1)
    r1 = 1.0 / (e10 + e11)
    attn = jnp.concatenate([e00 * r0, e10 * r1, e01 * r0, e11 * r1], axis=-1)

    wm = (x2.reshape(NI, HW, 2 * C) * attn.reshape(NI, 1, 2 * C)).reshape(M, 2 * C)
    s = jnp.maximum(wm[:, :C] + wm[:, C:], 0.0)
    out = jnp.maximum(s + res, 0.0)
    o_ref[...] = out.reshape(NI, HW, C).astype(o_ref.dtype)


def kernel(x, w_cat, s_cat, b_cat, m_cat, w2, s2, b2, m2, wfc1, sf1, bf1,
           wfc2, bfc2):
    B, Cin, H, W = x.shape
    C = w_cat.shape[1] // 2
    HW = H * W
    NI = 8 if B % 8 == 0 else (4 if B % 4 == 0 else 1)

    x3 = jnp.transpose(x, (0, 2, 3, 1)).reshape(B, HW, Cin).astype(jnp.bfloat16)
    wb1 = w_cat.astype(jnp.bfloat16)
    wb2 = w2.astype(jnp.bfloat16)

    kern = functools.partial(_block_kernel, NI=NI, H=H, W=W, C=C)
    const = lambda *_: (0, 0)
    out = pl.pallas_call(
        kern,
        out_shape=jax.ShapeDtypeStruct((B, HW, C), jnp.float32),
        grid=(B // NI,),
        in_specs=[
            pl.BlockSpec((NI, HW, Cin), lambda b: (b, 0, 0)),
            pl.BlockSpec(wb1.shape, const),
            pl.BlockSpec(s_cat.shape, const),
            pl.BlockSpec(b_cat.shape, const),
            pl.BlockSpec(m_cat.shape, const),
            pl.BlockSpec(wb2.shape, const),
            pl.BlockSpec(s2.shape, const),
            pl.BlockSpec(b2.shape, const),
            pl.BlockSpec(m2.shape, const),
            pl.BlockSpec(wfc1.shape, const),
            pl.BlockSpec(sf1.shape, const),
            pl.BlockSpec(bf1.shape, const),
            pl.BlockSpec(wfc2.shape, const),
            pl.BlockSpec(bfc2.shape, const),
        ],
        out_specs=pl.BlockSpec((NI, HW, C), lambda b: (b, 0, 0)),
        compiler_params=pltpu.CompilerParams(
            dimension_semantics=("arbitrary",)),
    )(x3, wb1, s_cat, b_cat, m_cat, wb2, s2, b2, m2, wfc1, sf1, bf1,
      wfc2, bfc2)

    return jnp.transpose(out.reshape(B, H, W, C), (0, 3, 1, 2))


# speedup vs baseline: 1.0551x; 1.0551x over previous
import functools

import jax
import jax.numpy as jnp
from jax import lax
from jax.experimental import pallas as pl
from jax.experimental.pallas import tpu as pltpu


def _im2col9(xf, NI, H, W):
    M, Cin = xf.shape
    HW = H * W
    dt = xf.dtype
    p = lax.broadcasted_iota(jnp.int32, (M, 1), 0)
    j = p % W
    pin = p % HW
    taps = []
    for kh in range(3):
        for kw in range(3):
            t = (kh - 1) * W + (kw - 1)
            if t > 0:
                sh = jnp.concatenate([xf[t:], jnp.zeros((t, Cin), dt)], axis=0)
            elif t < 0:
                sh = jnp.concatenate([jnp.zeros((-t, Cin), dt), xf[:t]], axis=0)
            else:
                sh = xf
            mask = None
            if kw == 0:
                mask = j != 0
            elif kw == 2:
                mask = j != W - 1
            if kh == 0:
                vm = pin >= W
                mask = vm if mask is None else (mask & vm)
            elif kh == 2:
                vm = pin < HW - W
                mask = vm if mask is None else (mask & vm)
            if mask is not None:
                sh = jnp.where(mask, sh, jnp.zeros((), dt))
            taps.append(sh)
    return jnp.concatenate(taps, axis=-1)


def _block_kernel(x_ref, w1_ref, s1_ref, b1_ref, mc_ref, w2_ref, s2_ref,
                  b2_ref, m2_ref, wfc1_ref, sf1_ref, bf1_ref, wfc2_ref,
                  bfc2_ref, o_ref, *, NI, H, W, C):
    HW = H * W
    M = NI * HW
    Cq = C // 2
    xf = x_ref[...].reshape(M, x_ref.shape[-1])

    TM = 1024
    w1b = w1_ref[...].astype(jnp.bfloat16)
    w2b = w2_ref[...].astype(jnp.bfloat16)
    cc1 = _im2col9(xf, NI, H, W)
    acc1 = jnp.concatenate(
        [jnp.dot(cc1[t:t + TM], w1b, preferred_element_type=jnp.float32)
         for t in range(0, M, TM)], axis=0)
    y = acc1 * s1_ref[...] + b1_ref[...]
    y = jnp.where(mc_ref[...] > 0.0, jnp.maximum(y, 0.0), y)
    y1 = y[:, :C].astype(jnp.bfloat16)
    res = y[:, C:]

    cc2 = _im2col9(y1, NI, H, W)
    acc2 = jnp.concatenate(
        [jnp.dot(cc2[t:t + TM], w2b, preferred_element_type=jnp.float32)
         for t in range(0, M, TM)], axis=0)
    x2 = acc2 * s2_ref[...] + b2_ref[...]
    x2 = jnp.where(m2_ref[...] > 0.0, jnp.maximum(x2, 0.0), x2)

    gsum = jnp.sum(x2.reshape(NI, HW, 2 * C), axis=1)
    gap = (gsum[:, :C] + gsum[:, C:]) * (1.0 / HW)
    g1 = jnp.dot(gap, wfc1_ref[...], preferred_element_type=jnp.float32)
    g1 = jnp.maximum(g1 * sf1_ref[...] + bf1_ref[...], 0.0)
    a = jnp.dot(g1, wfc2_ref[...], preferred_element_type=jnp.float32)
    a = a + bfc2_ref[...]

    a00, a01 = a[:, 0:Cq], a[:, Cq:2 * Cq]
    a10, a11 = a[:, 2 * Cq:3 * Cq], a[:, 3 * Cq:4 * Cq]
    m0 = jnp.maximum(a00, a01)
    e00, e01 = jnp.exp(a00 - m0), jnp.exp(a01 - m0)
    r0 = 1.0 / (e00 + e01)
    m1 = jnp.maximum(a10, a11)
    e10, e11 = jnp.exp(a10 - m1), jnp.exp(a11 - m1)
    r1 = 1.0 / (e10 + e11)
    attn = jnp.concatenate([e00 * r0, e10 * r1, e01 * r0, e11 * r1], axis=-1)

    wm = (x2.reshape(NI, HW, 2 * C) * attn.reshape(NI, 1, 2 * C)).reshape(M, 2 * C)
    s = jnp.maximum(wm[:, :C] + wm[:, C:], 0.0)
    out = jnp.maximum(s + res, 0.0)
    o_ref[...] = out.reshape(NI, HW, C).astype(o_ref.dtype)


def kernel(x, w_cat, s_cat, b_cat, m_cat, w2, s2, b2, m2, wfc1, sf1, bf1,
           wfc2, bfc2):
    B, Cin, H, W = x.shape
    C = w_cat.shape[1] // 2
    HW = H * W
    NI = 8 if B % 8 == 0 else (4 if B % 4 == 0 else 1)

    x3 = jnp.transpose(x, (0, 2, 3, 1)).reshape(B, HW, Cin).astype(jnp.bfloat16)
    wb1 = w_cat
    wb2 = w2

    kern = functools.partial(_block_kernel, NI=NI, H=H, W=W, C=C)
    const = lambda *_: (0, 0)
    out = pl.pallas_call(
        kern,
        out_shape=jax.ShapeDtypeStruct((B, HW, C), jnp.float32),
        grid=(B // NI,),
        in_specs=[
            pl.BlockSpec((NI, HW, Cin), lambda b: (b, 0, 0)),
            pl.BlockSpec(wb1.shape, const),
            pl.BlockSpec(s_cat.shape, const),
            pl.BlockSpec(b_cat.shape, const),
            pl.BlockSpec(m_cat.shape, const),
            pl.BlockSpec(wb2.shape, const),
            pl.BlockSpec(s2.shape, const),
            pl.BlockSpec(b2.shape, const),
            pl.BlockSpec(m2.shape, const),
            pl.BlockSpec(wfc1.shape, const),
            pl.BlockSpec(sf1.shape, const),
            pl.BlockSpec(bf1.shape, const),
            pl.BlockSpec(wfc2.shape, const),
            pl.BlockSpec(bfc2.shape, const),
        ],
        out_specs=pl.BlockSpec((NI, HW, C), lambda b: (b, 0, 0)),
        compiler_params=pltpu.CompilerParams(
            dimension_semantics=("arbitrary",)),
    )(x3, wb1, s_cat, b_cat, m_cat, wb2, s2, b2, m2, wfc1, sf1, bf1,
      wfc2, bfc2)

    return jnp.transpose(out.reshape(B, H, W, C), (0, 3, 1, 2))
